# baseline (speedup 1.0000x reference)
import functools

import jax
import jax.numpy as jnp
from jax.experimental import pallas as pl
from jax.experimental.pallas import tpu as pltpu
from jax.experimental.pallas import tpu_sc as plsc

DIM = 64
HIDDEN = 128

SC_CORES = 2
SC_SUBCORES = 16

GATHER_WINDOW = 128


def _sc_gather(table, idx_flat):
    n = idx_flat.shape[1]
    mesh = plsc.VectorSubcoreMesh(core_axis_name="c", subcore_axis_name="s")

    @functools.partial(
        pl.kernel,
        out_type=jax.ShapeDtypeStruct((n, DIM), table.dtype),
        mesh=mesh,
        compiler_params=pltpu.CompilerParams(use_tc_tiling_on_sc=False),
    )
    def gather_kernel(table_hbm, idx_hbm, out_hbm):
        def body(idx_vmem, out_vmem):
            pltpu.sync_copy(table_hbm.at[idx_vmem.at[0]], out_vmem)

        pltpu.emit_pipeline(
            body,
            grid=(n // GATHER_WINDOW,),
            in_specs=[
                pl.BlockSpec((1, GATHER_WINDOW), index_map=lambda i: (0, i))
            ],
            out_specs=[
                pl.BlockSpec((GATHER_WINDOW, DIM), index_map=lambda i: (i, 0))
            ],
            core_axis_name=("c", "s"),
            dimension_semantics=(pltpu.PARALLEL,),
        )(idx_hbm, out_hbm)

    return gather_kernel(table, idx_flat)


TC_ROWS = 1024


def _proj_body(emb_ref, w_ref, out_ref):
    h = jax.lax.dot_general(
        emb_ref[...],
        w_ref[...],
        (((1,), (1,)), ((), ())),
        preferred_element_type=jnp.float32,
    )
    ss = jnp.sum(h * h, axis=1, keepdims=True)
    out_ref[...] = h * jax.lax.rsqrt(jnp.maximum(ss, 1e-24))


def _tc_project(emb, w):
    n = emb.shape[0]
    return pl.pallas_call(
        _proj_body,
        grid=(n // TC_ROWS,),
        in_specs=[
            pl.BlockSpec((TC_ROWS, DIM), lambda i: (i, 0)),
            pl.BlockSpec((HIDDEN, DIM), lambda i: (0, 0)),
        ],
        out_specs=pl.BlockSpec((TC_ROWS, HIDDEN), lambda i: (i, 0)),
        out_shape=jax.ShapeDtypeStruct((n, HIDDEN), jnp.float32),
    )(emb, w)


def kernel(x, table, W):
    batch, fields = x.shape
    n = batch * fields
    idx_flat = x.reshape(1, n)
    emb = _sc_gather(table, idx_flat)
    out = _tc_project(emb, W)
    return out.reshape(batch, fields, HIDDEN)


# speedup vs baseline: 1.9644x; 1.9644x over previous
import functools

import jax
import jax.numpy as jnp
from jax.experimental import pallas as pl
from jax.experimental.pallas import tpu as pltpu
from jax.experimental.pallas import tpu_sc as plsc

DIM = 64
HIDDEN = 128

VOCAB_BLOCK = 2048
GATHER_WINDOW = 128


def _proj_body(tt_ref, wt_ref, out_ref):
    t = jnp.transpose(tt_ref[...])
    h = jax.lax.dot_general(
        t,
        wt_ref[...],
        (((1,), (0,)), ((), ())),
        preferred_element_type=jnp.float32,
        precision=jax.lax.Precision.HIGHEST,
    )
    ss = jnp.sum(h * h, axis=1, keepdims=True)
    out_ref[...] = h * jax.lax.rsqrt(jnp.maximum(ss, 1e-24))


def _tc_project_all(table_t, w_t):
    vocab = table_t.shape[1]
    return pl.pallas_call(
        _proj_body,
        grid=(pl.cdiv(vocab, VOCAB_BLOCK),),
        in_specs=[
            pl.BlockSpec((DIM, VOCAB_BLOCK), lambda i: (0, i)),
            pl.BlockSpec((DIM, HIDDEN), lambda i: (0, 0)),
        ],
        out_specs=pl.BlockSpec((VOCAB_BLOCK, HIDDEN), lambda i: (i, 0)),
        out_shape=jax.ShapeDtypeStruct((vocab, HIDDEN), jnp.float32),
    )(table_t, w_t)


def _sc_gather(rows, idx_flat):
    n = idx_flat.shape[1]
    mesh = plsc.VectorSubcoreMesh(core_axis_name="c", subcore_axis_name="s")

    @functools.partial(
        pl.kernel,
        out_type=jax.ShapeDtypeStruct((n, HIDDEN), rows.dtype),
        mesh=mesh,
    )
    def gather_kernel(rows_hbm, idx_hbm, out_hbm):
        def body(idx_vmem, out_vmem):
            pltpu.sync_copy(rows_hbm.at[idx_vmem.at[0]], out_vmem)

        pltpu.emit_pipeline(
            body,
            grid=(n // GATHER_WINDOW,),
            in_specs=[
                pl.BlockSpec((1, GATHER_WINDOW), index_map=lambda i: (0, i))
            ],
            out_specs=[
                pl.BlockSpec((GATHER_WINDOW, HIDDEN), index_map=lambda i: (i, 0))
            ],
            core_axis_name=("c", "s"),
            dimension_semantics=(pltpu.PARALLEL,),
        )(idx_hbm, out_hbm)

    return gather_kernel(rows, idx_flat)


def kernel(x, table, W):
    batch, fields = x.shape
    n = batch * fields
    hidden_norm = _tc_project_all(table.T, W.T)
    idx_flat = x.T.reshape(1, n)
    out = _sc_gather(hidden_norm, idx_flat)
    return out.reshape(fields, batch, HIDDEN).transpose(1, 0, 2)


# speedup vs baseline: 2.0235x; 1.0301x over previous
import functools

import jax
import jax.numpy as jnp
from jax.experimental import pallas as pl
from jax.experimental.pallas import tpu as pltpu
from jax.experimental.pallas import tpu_sc as plsc

DIM = 64
HIDDEN = 128

VOCAB_BLOCK = 2048
GATHER_WINDOW = 128


def _proj_body(tt_ref, wt_ref, out_ref):
    t = jnp.transpose(tt_ref[...])
    wt = wt_ref[...]

    def _split(a):
        hi = a.astype(jnp.bfloat16)
        lo = (a - hi.astype(jnp.float32)).astype(jnp.bfloat16)
        return hi, lo

    t_hi, t_lo = _split(t)
    w_hi, w_lo = _split(wt)
    dims = (((1,), (0,)), ((), ()))

    def _mm(a, b):
        return jax.lax.dot_general(a, b, dims,
                                   preferred_element_type=jnp.float32)

    h = _mm(t_hi, w_hi) + (_mm(t_lo, w_hi) + _mm(t_hi, w_lo))

    hb = h.astype(jnp.bfloat16)
    ss = jax.lax.dot_general(
        hb * hb,
        jnp.ones((HIDDEN, 8), jnp.bfloat16),
        dims,
        preferred_element_type=jnp.float32,
    )
    rinv = jax.lax.rsqrt(jnp.maximum(ss[:, :1], 1e-24))
    out_ref[...] = h * jax.lax.broadcast_in_dim(
        rinv, (h.shape[0], HIDDEN), (0, 1)
    )


def _tc_project_all(table_t, w_t):
    vocab = table_t.shape[1]
    return pl.pallas_call(
        _proj_body,
        grid=(pl.cdiv(vocab, VOCAB_BLOCK),),
        in_specs=[
            pl.BlockSpec((DIM, VOCAB_BLOCK), lambda i: (0, i)),
            pl.BlockSpec((DIM, HIDDEN), lambda i: (0, 0)),
        ],
        out_specs=pl.BlockSpec((VOCAB_BLOCK, HIDDEN), lambda i: (i, 0)),
        out_shape=jax.ShapeDtypeStruct((vocab, HIDDEN), jnp.float32),
    )(table_t, w_t)


def _sc_gather(rows, idx_flat):
    n = idx_flat.shape[1]
    mesh = plsc.VectorSubcoreMesh(core_axis_name="c", subcore_axis_name="s")

    @functools.partial(
        pl.kernel,
        out_type=jax.ShapeDtypeStruct((n, HIDDEN), rows.dtype),
        mesh=mesh,
    )
    def gather_kernel(rows_hbm, idx_hbm, out_hbm):
        def body(idx_vmem, out_vmem):
            pltpu.sync_copy(rows_hbm.at[idx_vmem.at[0]], out_vmem)

        pltpu.emit_pipeline(
            body,
            grid=(n // GATHER_WINDOW,),
            in_specs=[
                pl.BlockSpec((1, GATHER_WINDOW), index_map=lambda i: (0, i))
            ],
            out_specs=[
                pl.BlockSpec((GATHER_WINDOW, HIDDEN), index_map=lambda i: (i, 0))
            ],
            core_axis_name=("c", "s"),
            dimension_semantics=(pltpu.PARALLEL,),
        )(idx_hbm, out_hbm)

    return gather_kernel(rows, idx_flat)


def kernel(x, table, W):
    batch, fields = x.shape
    n = batch * fields
    hidden_norm = _tc_project_all(table.T, W.T)
    idx_flat = x.T.reshape(1, n)
    out = _sc_gather(hidden_norm, idx_flat)
    return out.reshape(fields, batch, HIDDEN).transpose(1, 0, 2)


# speedup vs baseline: 2.5599x; 1.2651x over previous
import functools

import jax
import jax.numpy as jnp
from jax.experimental import pallas as pl
from jax.experimental.pallas import tpu as pltpu
from jax.experimental.pallas import tpu_sc as plsc

DIM = 64
HIDDEN = 128

VOCAB_BLOCK = 4096
GATHER_WINDOW = 128


def _proj_body(tt_ref, wt_ref, out_ref):
    t = jnp.transpose(tt_ref[...])
    wt = wt_ref[...]

    def _split(a):
        hi = a.astype(jnp.bfloat16)
        lo = (a - hi.astype(jnp.float32)).astype(jnp.bfloat16)
        return hi, lo

    t_hi, t_lo = _split(t)
    w_hi, w_lo = _split(wt)
    dims = (((1,), (0,)), ((), ()))

    def _mm(a, b):
        return jax.lax.dot_general(a, b, dims,
                                   preferred_element_type=jnp.float32)

    h = _mm(t_hi, w_hi) + (_mm(t_lo, w_hi) + _mm(t_hi, w_lo))

    hb = h.astype(jnp.bfloat16)
    ss = jax.lax.dot_general(
        hb * hb,
        jnp.ones((HIDDEN, 8), jnp.bfloat16),
        dims,
        preferred_element_type=jnp.float32,
    )
    rinv = jax.lax.rsqrt(jnp.maximum(ss[:, :1], 1e-24))
    out_ref[...] = h * jax.lax.broadcast_in_dim(
        rinv, (h.shape[0], HIDDEN), (0, 1)
    )


def _tc_project_all(table_t, w_t):
    vocab = table_t.shape[1]
    return pl.pallas_call(
        _proj_body,
        grid=(pl.cdiv(vocab, VOCAB_BLOCK),),
        in_specs=[
            pl.BlockSpec((DIM, VOCAB_BLOCK), lambda i: (0, i)),
            pl.BlockSpec((DIM, HIDDEN), lambda i: (0, 0)),
        ],
        out_specs=pl.BlockSpec((VOCAB_BLOCK, HIDDEN), lambda i: (i, 0)),
        out_shape=jax.ShapeDtypeStruct((vocab, HIDDEN), jnp.float32),
    )(table_t, w_t)


def _sc_gather(rows, idx_flat):
    n = idx_flat.shape[1]
    mesh = plsc.VectorSubcoreMesh(core_axis_name="c", subcore_axis_name="s")

    @functools.partial(
        pl.kernel,
        out_type=jax.ShapeDtypeStruct((n, HIDDEN), rows.dtype),
        mesh=mesh,
    )
    def gather_kernel(rows_hbm, idx_hbm, out_hbm):
        def body(idx_vmem, out_vmem):
            pltpu.sync_copy(rows_hbm.at[idx_vmem.at[0]], out_vmem)

        pltpu.emit_pipeline(
            body,
            grid=(n // GATHER_WINDOW,),
            in_specs=[
                pl.BlockSpec((1, GATHER_WINDOW), index_map=lambda i: (0, i))
            ],
            out_specs=[
                pl.BlockSpec((GATHER_WINDOW, HIDDEN), index_map=lambda i: (i, 0))
            ],
            core_axis_name=("c", "s"),
            dimension_semantics=(pltpu.PARALLEL,),
        )(idx_hbm, out_hbm)

    return gather_kernel(rows, idx_flat)


def kernel(x, table, W):
    batch, fields = x.shape
    n = batch * fields
    hidden_norm = _tc_project_all(table.T, W.T)
    idx_flat = x.T.reshape(1, n)
    out = _sc_gather(hidden_norm, idx_flat)
    return out.reshape(fields, batch, HIDDEN).transpose(1, 0, 2)


# speedup vs baseline: 2.9844x; 1.1658x over previous
import functools

import jax
import jax.numpy as jnp
from jax.experimental import pallas as pl
from jax.experimental.pallas import tpu as pltpu
from jax.experimental.pallas import tpu_sc as plsc

DIM = 64
HIDDEN = 128

VOCAB_BLOCK = 8192
GATHER_WINDOW = 128


def _proj_body(tt_ref, wt_ref, out_ref):
    t = jnp.transpose(tt_ref[...])
    wt = wt_ref[...]

    def _split(a):
        hi = a.astype(jnp.bfloat16)
        lo = (a - hi.astype(jnp.float32)).astype(jnp.bfloat16)
        return hi, lo

    t_hi, t_lo = _split(t)
    w_hi, w_lo = _split(wt)
    dims = (((1,), (0,)), ((), ()))

    def _mm(a, b):
        return jax.lax.dot_general(a, b, dims,
                                   preferred_element_type=jnp.float32)

    h = _mm(t_hi, w_hi) + (_mm(t_lo, w_hi) + _mm(t_hi, w_lo))

    hb = h.astype(jnp.bfloat16)
    ss = jax.lax.dot_general(
        hb * hb,
        jnp.ones((HIDDEN, 8), jnp.bfloat16),
        dims,
        preferred_element_type=jnp.float32,
    )
    rinv = jax.lax.rsqrt(jnp.maximum(ss[:, :1], 1e-24))
    out_ref[...] = h * jax.lax.broadcast_in_dim(
        rinv, (h.shape[0], HIDDEN), (0, 1)
    )


def _tc_project_all(table_t, w_t):
    vocab = table_t.shape[1]
    return pl.pallas_call(
        _proj_body,
        grid=(pl.cdiv(vocab, VOCAB_BLOCK),),
        in_specs=[
            pl.BlockSpec((DIM, VOCAB_BLOCK), lambda i: (0, i)),
            pl.BlockSpec((DIM, HIDDEN), lambda i: (0, 0)),
        ],
        out_specs=pl.BlockSpec((VOCAB_BLOCK, HIDDEN), lambda i: (i, 0)),
        out_shape=jax.ShapeDtypeStruct((vocab, HIDDEN), jnp.float32),
    )(table_t, w_t)


def _sc_gather(rows, idx_flat):
    n = idx_flat.shape[1]
    mesh = plsc.VectorSubcoreMesh(core_axis_name="c", subcore_axis_name="s")

    @functools.partial(
        pl.kernel,
        out_type=jax.ShapeDtypeStruct((n, HIDDEN), rows.dtype),
        mesh=mesh,
    )
    def gather_kernel(rows_hbm, idx_hbm, out_hbm):
        def body(idx_vmem, out_vmem):
            pltpu.sync_copy(rows_hbm.at[idx_vmem.at[0]], out_vmem)

        pltpu.emit_pipeline(
            body,
            grid=(n // GATHER_WINDOW,),
            in_specs=[
                pl.BlockSpec((1, GATHER_WINDOW), index_map=lambda i: (0, i))
            ],
            out_specs=[
                pl.BlockSpec((GATHER_WINDOW, HIDDEN), index_map=lambda i: (i, 0))
            ],
            core_axis_name=("c", "s"),
            dimension_semantics=(pltpu.PARALLEL,),
        )(idx_hbm, out_hbm)

    return gather_kernel(rows, idx_flat)


def kernel(x, table, W):
    batch, fields = x.shape
    n = batch * fields
    hidden_norm = _tc_project_all(table.T, W.T)
    idx_flat = x.T.reshape(1, n)
    out = _sc_gather(hidden_norm, idx_flat)
    return out.reshape(fields, batch, HIDDEN).transpose(1, 0, 2)


# speedup vs baseline: 3.2023x; 1.0730x over previous
import functools

import jax
import jax.numpy as jnp
from jax.experimental import pallas as pl
from jax.experimental.pallas import tpu as pltpu
from jax.experimental.pallas import tpu_sc as plsc

DIM = 64
HIDDEN = 128

VOCAB_BLOCK = 16384
GATHER_WINDOW = 128


def _proj_body(tt_ref, wt_ref, out_ref):
    t = jnp.transpose(tt_ref[...])
    wt = wt_ref[...]

    def _split(a):
        hi = a.astype(jnp.bfloat16)
        lo = (a - hi.astype(jnp.float32)).astype(jnp.bfloat16)
        return hi, lo

    t_hi, t_lo = _split(t)
    w_hi, w_lo = _split(wt)
    dims = (((1,), (0,)), ((), ()))

    def _mm(a, b):
        return jax.lax.dot_general(a, b, dims,
                                   preferred_element_type=jnp.float32)

    h = _mm(t_hi, w_hi) + (_mm(t_lo, w_hi) + _mm(t_hi, w_lo))

    hb = h.astype(jnp.bfloat16)
    ss = jax.lax.dot_general(
        hb * hb,
        jnp.ones((HIDDEN, 8), jnp.bfloat16),
        dims,
        preferred_element_type=jnp.float32,
    )
    rinv = jax.lax.rsqrt(jnp.maximum(ss[:, :1], 1e-24))
    out_ref[...] = h * jax.lax.broadcast_in_dim(
        rinv, (h.shape[0], HIDDEN), (0, 1)
    )


def _tc_project_all(table_t, w_t):
    vocab = table_t.shape[1]
    return pl.pallas_call(
        _proj_body,
        grid=(pl.cdiv(vocab, VOCAB_BLOCK),),
        in_specs=[
            pl.BlockSpec((DIM, VOCAB_BLOCK), lambda i: (0, i)),
            pl.BlockSpec((DIM, HIDDEN), lambda i: (0, 0)),
        ],
        out_specs=pl.BlockSpec((VOCAB_BLOCK, HIDDEN), lambda i: (i, 0)),
        out_shape=jax.ShapeDtypeStruct((vocab, HIDDEN), jnp.float32),
    )(table_t, w_t)


def _sc_gather(rows, idx_flat):
    n = idx_flat.shape[1]
    mesh = plsc.VectorSubcoreMesh(core_axis_name="c", subcore_axis_name="s")

    @functools.partial(
        pl.kernel,
        out_type=jax.ShapeDtypeStruct((n, HIDDEN), rows.dtype),
        mesh=mesh,
    )
    def gather_kernel(rows_hbm, idx_hbm, out_hbm):
        def body(idx_vmem, out_vmem):
            pltpu.sync_copy(rows_hbm.at[idx_vmem.at[0]], out_vmem)

        pltpu.emit_pipeline(
            body,
            grid=(n // GATHER_WINDOW,),
            in_specs=[
                pl.BlockSpec((1, GATHER_WINDOW), index_map=lambda i: (0, i))
            ],
            out_specs=[
                pl.BlockSpec((GATHER_WINDOW, HIDDEN), index_map=lambda i: (i, 0))
            ],
            core_axis_name=("c", "s"),
            dimension_semantics=(pltpu.PARALLEL,),
        )(idx_hbm, out_hbm)

    return gather_kernel(rows, idx_flat)


def kernel(x, table, W):
    batch, fields = x.shape
    n = batch * fields
    hidden_norm = _tc_project_all(table.T, W.T)
    idx_flat = x.T.reshape(1, n)
    out = _sc_gather(hidden_norm, idx_flat)
    return out.reshape(fields, batch, HIDDEN).transpose(1, 0, 2)
